# Initial kernel scaffold; baseline (speedup 1.0000x reference)
import jax
import jax.numpy as jnp
from jax.experimental import pallas as pl


def kernel(query, keys, values):
    raise NotImplementedError("write your pallas kernel here")


# speedup vs baseline: 3.2180x; 3.2180x over previous
import functools

import jax
import jax.numpy as jnp
from jax import lax
from jax.experimental import pallas as pl
from jax.experimental.pallas import tpu as pltpu
from jax.experimental.pallas import tpu_sc as plsc

_N = 128
_L = 20
_B = 64
_DK = 512
_DV = 512
_TOPK = 8

_NBLK = 16
_GRID = _N // _NBLK

_ROWS = _TOPK * _B * _L
_NW = 32
_RPW = _ROWS // _NW
_CH = 64
_NCH = _RPW // _CH


def _topk_body(q_ref, k_ref, w_ref, rows_ref, s_ref):
    i = pl.program_id(0)
    q = q_ref[...].astype(jnp.bfloat16).astype(jnp.float32)
    kb = k_ref[...].astype(jnp.bfloat16).astype(jnp.float32)
    s = jnp.sum(kb * q[None, :, :], axis=-1)
    s_ref[pl.ds(i * _NBLK, _NBLK), :] = s / jnp.sqrt(jnp.float32(_DK))

    @pl.when(i == _GRID - 1)
    def _():
        scores = s_ref[...]
        m = jnp.max(scores, axis=0, keepdims=True)
        e = jnp.exp(scores - m)
        att = e / jnp.sum(e, axis=0, keepdims=True)
        iota = lax.broadcasted_iota(jnp.int32, (_N, _B), 0)
        cur = att
        idxs = []
        vals = []
        for _ in range(_TOPK):
            mv = jnp.max(cur, axis=0, keepdims=True)
            mi = jnp.min(jnp.where(cur == mv, iota, _N), axis=0, keepdims=True)
            idxs.append(mi)
            vals.append(mv)
            cur = jnp.where(iota == mi, -1.0, cur)
        idx = jnp.concatenate(idxs, axis=0)
        w = jnp.concatenate(vals, axis=0)
        wm = jnp.max(w, axis=0, keepdims=True)
        we = jnp.exp(w - wm)
        w_ref[...] = jnp.transpose(we / jnp.sum(we, axis=0, keepdims=True))
        l_iota = lax.broadcasted_iota(jnp.int32, (_TOPK, _B, _L), 2)
        b_iota = lax.broadcasted_iota(jnp.int32, (_TOPK, _B, _L), 1)
        rows_ref[...] = idx[:, :, None] * (_L * _B) + l_iota * _B + b_iota


def _topk_call(q, keys):
    return pl.pallas_call(
        _topk_body,
        grid=(_GRID,),
        in_specs=[
            pl.BlockSpec((_B, _DK), lambda i: (0, 0)),
            pl.BlockSpec((_NBLK, _B, _DK), lambda i: (i, 0, 0)),
        ],
        out_specs=[
            pl.BlockSpec((_B, _TOPK), lambda i: (0, 0)),
            pl.BlockSpec((_TOPK, _B, _L), lambda i: (0, 0, 0)),
        ],
        out_shape=[
            jax.ShapeDtypeStruct((_B, _TOPK), jnp.float32),
            jax.ShapeDtypeStruct((_TOPK, _B, _L), jnp.int32),
        ],
        scratch_shapes=[pltpu.VMEM((_N, _B), jnp.float32)],
    )(q, keys)


def _gather_body(rows_hbm, table_hbm, out_hbm, idx_v, buf0, buf1, sem0, sem1):
    wid = lax.axis_index("s") * 2 + lax.axis_index("c")
    pltpu.sync_copy(rows_hbm.at[wid], idx_v)
    bufs = (buf0, buf1)
    sems = (sem0, sem1)
    prev = pltpu.async_copy(table_hbm.at[idx_v.at[0]], bufs[0], sems[0])
    for c in range(_NCH):
        cur = prev
        if c + 1 < _NCH:
            prev = pltpu.async_copy(
                table_hbm.at[idx_v.at[c + 1]], bufs[(c + 1) % 2], sems[(c + 1) % 2])
        cur.wait()
        pltpu.sync_copy(bufs[c % 2], out_hbm.at[pl.ds(wid * _RPW + c * _CH, _CH)])


def _gather_call(rows, table):
    mesh = plsc.VectorSubcoreMesh(core_axis_name="c", subcore_axis_name="s")
    f = functools.partial(
        pl.kernel,
        mesh=mesh,
        out_type=jax.ShapeDtypeStruct((_ROWS, _DV), jnp.float32),
        scratch_types=[
            pltpu.VMEM((_NCH, _CH), jnp.int32),
            pltpu.VMEM((_CH, _DV), jnp.float32),
            pltpu.VMEM((_CH, _DV), jnp.float32),
            pltpu.SemaphoreType.DMA,
            pltpu.SemaphoreType.DMA,
        ],
    )(_gather_body)
    return f(rows, table)


def kernel(query, keys, values):
    q = query.reshape(_B, _DK)
    w, rows = _topk_call(q, keys)
    rows = rows.reshape(_NW, _NCH, _CH)
    table = values.reshape(_N * _L * _B, _DV)
    out = _gather_call(rows, table)
    topk_weights = w.reshape(_B, 1, _TOPK)
    outputs = out.reshape(_TOPK, _B, _L, _DV)
    return (topk_weights, outputs)


# speedup vs baseline: 6.3903x; 1.9858x over previous
import functools

import jax
import jax.numpy as jnp
from jax import lax
from jax.experimental import pallas as pl
from jax.experimental.pallas import tpu as pltpu
from jax.experimental.pallas import tpu_sc as plsc

_N = 128
_L = 20
_B = 64
_DK = 512
_DV = 512
_TOPK = 8

_NBLK = 16
_GRID = _N // _NBLK

_ROWS = _TOPK * _B * _L
_NW = 32
_RPW = _ROWS // _NW
_CH = 64
_NCH = _RPW // _CH


def _topk_body(q_ref, k_ref, w_ref, rows_ref, s_ref):
    i = pl.program_id(0)
    q = q_ref[...].astype(jnp.bfloat16).astype(jnp.float32)
    kb = k_ref[...].astype(jnp.bfloat16).astype(jnp.float32)
    s = jnp.sum(kb * q[None, :, :], axis=-1)
    s_ref[pl.ds(i * _NBLK, _NBLK), :] = s / jnp.sqrt(jnp.float32(_DK))

    @pl.when(i == _GRID - 1)
    def _():
        scores = s_ref[...]
        m = jnp.max(scores, axis=0, keepdims=True)
        e = jnp.exp(scores - m)
        att = e / jnp.sum(e, axis=0, keepdims=True)
        iota = lax.broadcasted_iota(jnp.int32, (_N, _B), 0)
        cur = att
        idxs = []
        vals = []
        for _ in range(_TOPK):
            mv = jnp.max(cur, axis=0, keepdims=True)
            mi = jnp.min(jnp.where(cur == mv, iota, _N), axis=0, keepdims=True)
            idxs.append(mi)
            vals.append(mv)
            cur = jnp.where(iota == mi, -1.0, cur)
        idx = jnp.concatenate(idxs, axis=0)
        w = jnp.concatenate(vals, axis=0)
        wm = jnp.max(w, axis=0, keepdims=True)
        we = jnp.exp(w - wm)
        w_ref[...] = we / jnp.sum(we, axis=0, keepdims=True)
        l_iota = lax.broadcasted_iota(jnp.int32, (_TOPK, _L, _B), 1)
        b_iota = lax.broadcasted_iota(jnp.int32, (_TOPK, _L, _B), 2)
        rows_ref[...] = idx[:, None, :] * (_L * _B) + l_iota * _B + b_iota


def _topk_call(q, keys):
    return pl.pallas_call(
        _topk_body,
        grid=(_GRID,),
        in_specs=[
            pl.BlockSpec((_B, _DK), lambda i: (0, 0)),
            pl.BlockSpec((_NBLK, _B, _DK), lambda i: (i, 0, 0)),
        ],
        out_specs=[
            pl.BlockSpec((_TOPK, _B), lambda i: (0, 0)),
            pl.BlockSpec((_TOPK, _L, _B), lambda i: (0, 0, 0)),
        ],
        out_shape=[
            jax.ShapeDtypeStruct((_TOPK, _B), jnp.float32),
            jax.ShapeDtypeStruct((_TOPK, _L, _B), jnp.int32),
        ],
        scratch_shapes=[pltpu.VMEM((_N, _B), jnp.float32)],
    )(q, keys)


def _gather_body(rows_hbm, table_hbm, out_hbm, idx_v, buf0, buf1, sem0, sem1):
    wid = lax.axis_index("s") * 2 + lax.axis_index("c")
    pltpu.sync_copy(rows_hbm.at[wid], idx_v)
    bufs = (buf0, buf1)
    sems = (sem0, sem1)
    prev = pltpu.async_copy(table_hbm.at[idx_v.at[0]], bufs[0], sems[0])
    for c in range(_NCH):
        cur = prev
        if c + 1 < _NCH:
            prev = pltpu.async_copy(
                table_hbm.at[idx_v.at[c + 1]], bufs[(c + 1) % 2], sems[(c + 1) % 2])
        cur.wait()
        pltpu.sync_copy(bufs[c % 2], out_hbm.at[pl.ds(wid * _RPW + c * _CH, _CH)])


def _gather_call(rows, table):
    mesh = plsc.VectorSubcoreMesh(core_axis_name="c", subcore_axis_name="s")
    f = functools.partial(
        pl.kernel,
        mesh=mesh,
        out_type=jax.ShapeDtypeStruct((_ROWS, _DV), jnp.float32),
        scratch_types=[
            pltpu.VMEM((_NCH, _CH), jnp.int32),
            pltpu.VMEM((_CH, _DV), jnp.float32),
            pltpu.VMEM((_CH, _DV), jnp.float32),
            pltpu.SemaphoreType.DMA,
            pltpu.SemaphoreType.DMA,
        ],
    )(_gather_body)
    return f(rows, table)


def kernel(query, keys, values):
    q = query.reshape(_B, _DK)
    w, rows = _topk_call(q, keys)
    rows = rows.reshape(_NW, _NCH, _CH)
    table = values.reshape(_N * _L * _B, _DV)
    out = _gather_call(rows, table)
    topk_weights = jnp.transpose(w).reshape(_B, 1, _TOPK)
    outputs = jnp.swapaxes(out.reshape(_TOPK, _L, _B, _DV), 1, 2)
    return (topk_weights, outputs)
